# baseline (speedup 1.0000x reference)
import functools

import jax
import jax.numpy as jnp
from jax.experimental import pallas as pl
from jax.experimental.pallas import tpu as pltpu

_N = 4096
_D = 2048
_R = 64
_E = 8
_ALPHA = 2.0
_C = (_E + 1) * _R
_BN = 512
_G = _N // _BN


def _moe_lora_body(lab_ref, x1_ref, x2_ref, a_ref, b_ref, o_ref, h_scr):
    i = pl.program_id(0)
    rd = jax.lax.rem(i + 1, 2)
    wr = jax.lax.rem(i, 2)
    out = jax.lax.dot_general(
        h_scr[rd], b_ref[...], (((1,), (0,)), ((), ())),
        preferred_element_type=jnp.float32,
    )
    o_ref[...] = out
    xa = x1_ref[...].astype(jnp.bfloat16)
    xb = x2_ref[...].astype(jnp.bfloat16)
    h = jax.lax.dot_general(
        xa, a_ref[:, 0:_D // 2], (((1,), (1,)), ((), ())),
        preferred_element_type=jnp.float32,
    ) + jax.lax.dot_general(
        xb, a_ref[:, _D // 2:_D], (((1,), (1,)), ((), ())),
        preferred_element_type=jnp.float32,
    )
    lab = lab_ref[...]
    col = jax.lax.broadcasted_iota(jnp.int32, h.shape, 1)
    row = jax.lax.broadcasted_iota(jnp.int32, h.shape, 0) + i * _BN
    keep = ((col // _R == lab) | (col >= _E * _R)) & (row != _N - 1)
    h_scr[wr] = jnp.where(keep, h * _ALPHA, 0.0).astype(jnp.bfloat16)


@functools.partial(jax.jit, static_argnames=())
def kernel(x, label, weight, A_gen, B_gen, A_spec, B_spec):
    del weight
    lab = label.astype(jnp.int32).reshape(_N, 1)
    a_cat = jnp.concatenate(
        [A_spec.reshape(_E * _R, _D), A_gen], axis=0).astype(jnp.bfloat16)
    b_cat = jnp.concatenate(
        [B_spec.transpose(0, 2, 1).reshape(_E * _R, _D), B_gen.T],
        axis=0).astype(jnp.bfloat16)
    return pl.pallas_call(
        _moe_lora_body,
        grid=(_G + 1,),
        in_specs=[
            pl.BlockSpec((_BN, 1), lambda i: (jnp.minimum(i, _G - 1), 0)),
            pl.BlockSpec((_BN, _D // 2), lambda i: (jnp.minimum(i, _G - 1), 0)),
            pl.BlockSpec((_BN, _D // 2), lambda i: (jnp.minimum(i, _G - 1), 1)),
            pl.BlockSpec((_C, _D), lambda i: (0, 0)),
            pl.BlockSpec((_C, _D), lambda i: (0, 0)),
        ],
        out_specs=pl.BlockSpec((_BN, _D), lambda i: (jnp.maximum(i - 1, 0), 0)),
        out_shape=jax.ShapeDtypeStruct((_N, _D), jnp.float32),
        scratch_shapes=[
            pltpu.VMEM((2, _BN, _C), jnp.bfloat16),
        ],
    )(lab, x, x, a_cat, b_cat)


# speedup vs baseline: 1.0088x; 1.0088x over previous
import functools

import jax
import jax.numpy as jnp
from jax.experimental import pallas as pl
from jax.experimental.pallas import tpu as pltpu

_N = 4096
_D = 2048
_R = 64
_E = 8
_ALPHA = 2.0
_C = (_E + 1) * _R
_BN = 512
_G = _N // _BN


def _moe_lora_body(lab_ref, x_ref, a_ref, b_ref, o_ref, h_scr):
    i = pl.program_id(0)
    rd = jax.lax.rem(i + 1, 2)
    wr = jax.lax.rem(i, 2)
    out = jax.lax.dot_general(
        h_scr[rd], b_ref[...], (((1,), (0,)), ((), ())),
        preferred_element_type=jnp.float32,
    )
    o_ref[...] = out
    x = x_ref[...].astype(jnp.bfloat16)
    h = jax.lax.dot_general(
        x, a_ref[...], (((1,), (1,)), ((), ())),
        preferred_element_type=jnp.float32,
    )
    lab = lab_ref[...]
    col = jax.lax.broadcasted_iota(jnp.int32, h.shape, 1)
    row = jax.lax.broadcasted_iota(jnp.int32, h.shape, 0) + i * _BN
    keep = ((col // _R == lab) | (col >= _E * _R)) & (row != _N - 1)
    h_scr[wr] = jnp.where(keep, h * _ALPHA, 0.0).astype(jnp.bfloat16)


@functools.partial(jax.jit, static_argnames=())
def kernel(x, label, weight, A_gen, B_gen, A_spec, B_spec):
    del weight
    lab = label.astype(jnp.int32).reshape(_N, 1)
    a_cat = jnp.concatenate(
        [A_spec.reshape(_E * _R, _D), A_gen], axis=0).astype(jnp.bfloat16)
    b_cat = jnp.concatenate(
        [B_spec.transpose(0, 2, 1).reshape(_E * _R, _D), B_gen.T],
        axis=0).astype(jnp.bfloat16)
    return pl.pallas_call(
        _moe_lora_body,
        grid=(_G + 1,),
        in_specs=[
            pl.BlockSpec((_BN, 1), lambda i: (jnp.minimum(i, _G - 1), 0)),
            pl.BlockSpec((_BN, _D), lambda i: (jnp.minimum(i, _G - 1), 0)),
            pl.BlockSpec((_C, _D), lambda i: (0, 0)),
            pl.BlockSpec((_C, _D), lambda i: (0, 0)),
        ],
        out_specs=pl.BlockSpec((_BN, _D), lambda i: (jnp.maximum(i - 1, 0), 0)),
        out_shape=jax.ShapeDtypeStruct((_N, _D), jnp.float32),
        scratch_shapes=[
            pltpu.VMEM((2, _BN, _C), jnp.bfloat16),
        ],
    )(lab, x, a_cat, b_cat)
